# baseline (speedup 1.0000x reference)
import functools

import jax
import jax.numpy as jnp
from jax import lax
from jax.experimental import pallas as pl
from jax.experimental.pallas import tpu as pltpu
from jax.experimental.pallas import tpu_sc as plsc

NUM_POS = 50
NUM_BATCH = 16384
NUM_TOKENS = NUM_BATCH * NUM_POS
VOCAB = 1000000
EMBED_DIM = 32
NUM_CORES = 2
NUM_SUBCORES = 16
NUM_WORKERS = NUM_CORES * NUM_SUBCORES

FULL_TILES = VOCAB // 128
TPW = FULL_TILES // NUM_WORKERS
DBL_TILES = FULL_TILES // 2
T2PW = DBL_TILES // NUM_WORKERS
EXTRA_T2 = NUM_WORKERS * T2PW
TAIL_COLS = VOCAB - FULL_TILES * 128

IBLOCKS = NUM_BATCH // 128
UNITS = NUM_POS * IBLOCKS
UPW = UNITS // NUM_WORKERS
DUNITS = UNITS // 2
D2PW = DUNITS // NUM_WORKERS
IBP = IBLOCKS // 2

_mesh = plsc.VectorSubcoreMesh(core_axis_name="c", subcore_axis_name="s")


@functools.partial(
    pl.kernel,
    mesh=_mesh,
    compiler_params=pltpu.CompilerParams(needs_layout_passes=False),
    out_type=(
        jax.ShapeDtypeStruct((NUM_TOKENS,), jnp.int32),
        jax.ShapeDtypeStruct((VOCAB * EMBED_DIM,), jnp.float32),
    ),
    scratch_types=[
        pltpu.VMEM((8, 2048), jnp.int32),
        pltpu.VMEM((32, 256), jnp.float32),
        pltpu.VMEM((32, 256), jnp.float32),
        pltpu.VMEM((8192,), jnp.float32),
        pltpu.VMEM((8192,), jnp.float32),
        pltpu.SemaphoreType.DMA,
        pltpu.SemaphoreType.DMA,
        pltpu.SemaphoreType.DMA,
        pltpu.SemaphoreType.DMA,
    ],
)
def _prep_kernel(ids_hbm, tbl_hbm, tail_hbm, ids_out, tbl_out, idsbuf,
                 cb_a, cb_b, tb_a, tb_b, sem_ra, sem_rb, sem_wa, sem_wb):
    w = lax.axis_index("s") * NUM_CORES + lax.axis_index("c")
    iota = lax.iota(jnp.int32, 16)
    iota0 = iota * 0
    c_lo = iota
    c_hi = iota + 16

    @pl.when(w < 28)
    def _():
        for k in range(2):
            su = w * 2 + k
            jb = su // 8
            cc = su % 8
            pltpu.sync_copy(
                ids_hbm.at[pl.ds(jb * 8, 8), pl.ds(cc * 2048, 2048)], idsbuf)
            for r in range(8):
                @pl.when(jb * 8 + r < NUM_POS)
                def _():
                    pltpu.sync_copy(
                        idsbuf.at[r],
                        ids_out.at[pl.ds((jb * 8 + r) * NUM_BATCH + cc * 2048,
                                         2048)])

    def read_start(t, buf, sem):
        for cb in range(4):
            pltpu.async_copy(
                tbl_hbm.at[pl.ds(cb * 8, 8), pl.ds(t * 256, 256)],
                buf.at[pl.ds(cb * 8, 8)], sem)

    def read_wait(t, buf, sem):
        for cb in range(4):
            pltpu.make_async_copy(
                tbl_hbm.at[pl.ds(cb * 8, 8), pl.ds(t * 256, 256)],
                buf.at[pl.ds(cb * 8, 8)], sem).wait()

    def write_start(t, buf, sem):
        pltpu.async_copy(buf, tbl_out.at[pl.ds(t * 8192, 8192)], sem)

    def write_wait(t, buf, sem):
        pltpu.make_async_copy(
            buf, tbl_out.at[pl.ds(t * 8192, 8192)], sem).wait()

    def transpose_tile(src, dst):
        @plsc.parallel_loop(0, 256, unroll=8)
        def _(y):
            y_idx = iota0 + y
            dst[pl.ds(y * 32, 16)] = plsc.load_gather(src, [c_lo, y_idx])
            dst[pl.ds(y * 32 + 16, 16)] = plsc.load_gather(src, [c_hi, y_idx])

    t0 = w * T2PW
    NS = T2PW // 2
    read_start(t0, cb_a, sem_ra)

    def body(s, carry):
        ta = t0 + 2 * s
        tb = ta + 1
        read_wait(ta, cb_a, sem_ra)
        read_start(tb, cb_b, sem_rb)
        transpose_tile(cb_a, tb_a)

        @pl.when(s > 0)
        def _():
            write_wait(ta - 2, tb_a, sem_wa)

        write_start(ta, tb_a, sem_wa)
        read_wait(tb, cb_b, sem_rb)

        @pl.when(s < NS - 1)
        def _():
            read_start(ta + 2, cb_a, sem_ra)

        transpose_tile(cb_b, tb_b)

        @pl.when(s > 0)
        def _():
            write_wait(tb - 2, tb_b, sem_wb)

        write_start(tb, tb_b, sem_wb)
        return carry

    lax.fori_loop(0, NS, body, 0)
    write_wait(t0 + T2PW - 2, tb_a, sem_wa)
    write_wait(t0 + T2PW - 1, tb_b, sem_wb)

    @pl.when((w == 28) | (w == 29))
    def _():
        t = EXTRA_T2 + (w - 28)
        read_start(t, cb_a, sem_ra)
        read_wait(t, cb_a, sem_ra)
        transpose_tile(cb_a, tb_a)
        write_start(t, tb_a, sem_wa)
        write_wait(t, tb_a, sem_wa)

    @pl.when(w == 27)
    def _():
        pltpu.sync_copy(tail_hbm, tb_a.at[pl.ds(0, TAIL_COLS * EMBED_DIM)])
        pltpu.sync_copy(
            tb_a.at[pl.ds(0, TAIL_COLS * EMBED_DIM)],
            tbl_out.at[pl.ds(FULL_TILES * 128 * EMBED_DIM,
                             TAIL_COLS * EMBED_DIM)])


@functools.partial(
    pl.kernel,
    mesh=_mesh,
    compiler_params=pltpu.CompilerParams(
        use_tc_tiling_on_sc=False, needs_layout_passes=False),
    out_type=jax.ShapeDtypeStruct((NUM_POS, 4, IBLOCKS, 8, 128), jnp.float32),
    scratch_types=[
        pltpu.VMEM((UPW * 128,), jnp.int32),
        pltpu.VMEM((256, EMBED_DIM), jnp.float32),
        pltpu.VMEM((256, EMBED_DIM), jnp.float32),
        pltpu.VMEM((4, 2, 8, 128), jnp.float32),
        pltpu.VMEM((4, 2, 8, 128), jnp.float32),
        pltpu.SemaphoreType.DMA,
        pltpu.SemaphoreType.DMA,
        pltpu.SemaphoreType.DMA,
        pltpu.SemaphoreType.DMA,
    ],
)
def _gather_kernel(ids_hbm, tbl_hbm, out_hbm, idx_v, rows_a, rows_b, ob_a,
                   ob_b, sem_ga, sem_gb, sem_wa, sem_wb):
    w = lax.axis_index("s") * NUM_CORES + lax.axis_index("c")
    iota = lax.iota(jnp.int32, 16)
    iota0 = iota * 0
    r_base = [iota + 16 * ilb for ilb in range(16)]
    u0 = w * D2PW
    pltpu.sync_copy(ids_hbm.at[pl.ds(u0 * 256, D2PW * 256)], idx_v)

    def gather_start(u, buf, sem):
        pltpu.async_copy(
            tbl_hbm.at[idx_v.at[pl.ds((u - u0) * 256, 256)]], buf, sem)

    def gather_wait(u, buf, sem):
        pltpu.make_async_copy(
            tbl_hbm.at[idx_v.at[pl.ds((u - u0) * 256, 256)]], buf, sem).wait()

    def write_start(u, buf, sem):
        j = u // IBP
        ibp = u % IBP
        pltpu.async_copy(buf, out_hbm.at[j, :, pl.ds(ibp * 2, 2)], sem)

    def write_wait(u, buf, sem):
        j = u // IBP
        ibp = u % IBP
        pltpu.make_async_copy(
            buf, out_hbm.at[j, :, pl.ds(ibp * 2, 2)], sem).wait()

    def transpose_unit(src, dst):
        @plsc.parallel_loop(0, 32, unroll=4)
        def _(c):
            c_idx = iota0 + c
            cb = lax.shift_right_logical(c, 3)
            cr = c & 7
            for ilb in range(16):
                dst[cb, ilb // 8, cr, pl.ds((ilb % 8) * 16, 16)] = (
                    plsc.load_gather(src, [r_base[ilb], c_idx]))

    NS = D2PW // 2
    gather_start(u0, rows_a, sem_ga)

    def body(s, carry):
        ua = u0 + 2 * s
        ub = ua + 1
        gather_wait(ua, rows_a, sem_ga)
        gather_start(ub, rows_b, sem_gb)
        transpose_unit(rows_a, ob_a)

        @pl.when(s > 0)
        def _():
            write_wait(ua - 2, ob_a, sem_wa)

        write_start(ua, ob_a, sem_wa)
        gather_wait(ub, rows_b, sem_gb)

        @pl.when(s < NS - 1)
        def _():
            gather_start(ua + 2, rows_a, sem_ga)

        transpose_unit(rows_b, ob_b)

        @pl.when(s > 0)
        def _():
            write_wait(ub - 2, ob_b, sem_wb)

        write_start(ub, ob_b, sem_wb)
        return carry

    lax.fori_loop(0, NS, body, 0)
    write_wait(u0 + D2PW - 2, ob_a, sem_wa)
    write_wait(u0 + D2PW - 1, ob_b, sem_wb)


@jax.jit
def kernel(token_ids, embeddings):
    ids_t = token_ids.T.astype(jnp.int32)
    tbl_t = embeddings.T
    tail_flat = embeddings[FULL_TILES * 128:].reshape(-1)
    ids_lin, tbl_flat = _prep_kernel(ids_t, tbl_t, tail_flat)
    out5 = _gather_kernel(ids_lin, tbl_flat.reshape(VOCAB, EMBED_DIM))
    out = out5.transpose(2, 4, 0, 1, 3).reshape(NUM_BATCH, NUM_POS, EMBED_DIM)
    return out


# speedup vs baseline: 1.3831x; 1.3831x over previous
import functools

import jax
import jax.numpy as jnp
from jax import lax
from jax.experimental import pallas as pl
from jax.experimental.pallas import tpu as pltpu
from jax.experimental.pallas import tpu_sc as plsc

NUM_POS = 50
NUM_BATCH = 16384
NUM_TOKENS = NUM_BATCH * NUM_POS
VOCAB = 1000000
EMBED_DIM = 32
NUM_CORES = 2
NUM_SUBCORES = 16
NUM_WORKERS = NUM_CORES * NUM_SUBCORES

FULL_TILES = VOCAB // 128
TPW = FULL_TILES // NUM_WORKERS
DBL_TILES = FULL_TILES // 2
T2PW = DBL_TILES // NUM_WORKERS
EXTRA_T2 = NUM_WORKERS * T2PW
TAIL_COLS = VOCAB - FULL_TILES * 128

IBLOCKS = NUM_BATCH // 128
UNITS = NUM_POS * IBLOCKS
UPW = UNITS // NUM_WORKERS
DUNITS = UNITS // 2
D2PW = DUNITS // NUM_WORKERS
IBP = IBLOCKS // 2

_mesh = plsc.VectorSubcoreMesh(core_axis_name="c", subcore_axis_name="s")


@functools.partial(
    pl.kernel,
    mesh=_mesh,
    compiler_params=pltpu.CompilerParams(needs_layout_passes=False),
    out_type=(
        jax.ShapeDtypeStruct((NUM_TOKENS,), jnp.int32),
        jax.ShapeDtypeStruct((VOCAB * EMBED_DIM,), jnp.float32),
    ),
    scratch_types=[
        pltpu.VMEM((8, 2048), jnp.int32),
        pltpu.VMEM((32, 257), jnp.float32),
        pltpu.VMEM((32, 257), jnp.float32),
        pltpu.VMEM((8192,), jnp.float32),
        pltpu.VMEM((8192,), jnp.float32),
        pltpu.SemaphoreType.DMA,
        pltpu.SemaphoreType.DMA,
        pltpu.SemaphoreType.DMA,
        pltpu.SemaphoreType.DMA,
    ],
)
def _prep_kernel(ids_hbm, tbl_hbm, tail_hbm, ids_out, tbl_out, idsbuf,
                 cb_a, cb_b, tb_a, tb_b, sem_ra, sem_rb, sem_wa, sem_wb):
    w = lax.axis_index("s") * NUM_CORES + lax.axis_index("c")
    iota = lax.iota(jnp.int32, 16)
    iota0 = iota * 0
    c_lo = iota
    c_hi = iota + 16

    @pl.when(w < 28)
    def _():
        for k in range(2):
            su = w * 2 + k
            jb = su // 8
            cc = su % 8
            pltpu.sync_copy(
                ids_hbm.at[pl.ds(jb * 8, 8), pl.ds(cc * 2048, 2048)], idsbuf)
            for r in range(8):
                @pl.when(jb * 8 + r < NUM_POS)
                def _():
                    pltpu.sync_copy(
                        idsbuf.at[r],
                        ids_out.at[pl.ds((jb * 8 + r) * NUM_BATCH + cc * 2048,
                                         2048)])

    def read_start(t, buf, sem):
        for cb in range(4):
            pltpu.async_copy(
                tbl_hbm.at[pl.ds(cb * 8, 8), pl.ds(t * 256, 256)],
                buf.at[pl.ds(cb * 8, 8), pl.ds(0, 256)], sem)

    def read_wait(t, buf, sem):
        for cb in range(4):
            pltpu.make_async_copy(
                tbl_hbm.at[pl.ds(cb * 8, 8), pl.ds(t * 256, 256)],
                buf.at[pl.ds(cb * 8, 8), pl.ds(0, 256)], sem).wait()

    def write_start(t, buf, sem):
        pltpu.async_copy(buf, tbl_out.at[pl.ds(t * 8192, 8192)], sem)

    def write_wait(t, buf, sem):
        pltpu.make_async_copy(
            buf, tbl_out.at[pl.ds(t * 8192, 8192)], sem).wait()

    def transpose_tile(src, dst):
        @plsc.parallel_loop(0, 256, unroll=8)
        def _(y):
            y_idx = iota0 + y
            dst[pl.ds(y * 32, 16)] = plsc.load_gather(src, [c_lo, y_idx])
            dst[pl.ds(y * 32 + 16, 16)] = plsc.load_gather(src, [c_hi, y_idx])

    t0 = w * T2PW
    NS = T2PW // 2
    read_start(t0, cb_a, sem_ra)

    def body(s, carry):
        ta = t0 + 2 * s
        tb = ta + 1
        read_wait(ta, cb_a, sem_ra)
        read_start(tb, cb_b, sem_rb)
        transpose_tile(cb_a, tb_a)

        @pl.when(s > 0)
        def _():
            write_wait(ta - 2, tb_a, sem_wa)

        write_start(ta, tb_a, sem_wa)
        read_wait(tb, cb_b, sem_rb)

        @pl.when(s < NS - 1)
        def _():
            read_start(ta + 2, cb_a, sem_ra)

        transpose_tile(cb_b, tb_b)

        @pl.when(s > 0)
        def _():
            write_wait(tb - 2, tb_b, sem_wb)

        write_start(tb, tb_b, sem_wb)
        return carry

    lax.fori_loop(0, NS, body, 0)
    write_wait(t0 + T2PW - 2, tb_a, sem_wa)
    write_wait(t0 + T2PW - 1, tb_b, sem_wb)

    @pl.when((w == 28) | (w == 29))
    def _():
        t = EXTRA_T2 + (w - 28)
        read_start(t, cb_a, sem_ra)
        read_wait(t, cb_a, sem_ra)
        transpose_tile(cb_a, tb_a)
        write_start(t, tb_a, sem_wa)
        write_wait(t, tb_a, sem_wa)

    @pl.when(w == 27)
    def _():
        pltpu.sync_copy(tail_hbm, tb_a.at[pl.ds(0, TAIL_COLS * EMBED_DIM)])
        pltpu.sync_copy(
            tb_a.at[pl.ds(0, TAIL_COLS * EMBED_DIM)],
            tbl_out.at[pl.ds(FULL_TILES * 128 * EMBED_DIM,
                             TAIL_COLS * EMBED_DIM)])


@functools.partial(
    pl.kernel,
    mesh=_mesh,
    compiler_params=pltpu.CompilerParams(
        use_tc_tiling_on_sc=False, needs_layout_passes=False),
    out_type=jax.ShapeDtypeStruct((NUM_POS, 4, IBLOCKS, 8, 128), jnp.float32),
    scratch_types=[
        pltpu.VMEM((UPW * 128,), jnp.int32),
        pltpu.VMEM((256, EMBED_DIM), jnp.float32),
        pltpu.VMEM((256, EMBED_DIM), jnp.float32),
        pltpu.VMEM((4, 3, 8, 129), jnp.float32),
        pltpu.VMEM((4, 3, 8, 129), jnp.float32),
        pltpu.SemaphoreType.DMA,
        pltpu.SemaphoreType.DMA,
        pltpu.SemaphoreType.DMA,
        pltpu.SemaphoreType.DMA,
    ],
)
def _gather_kernel(ids_hbm, tbl_hbm, out_hbm, idx_v, rows_a, rows_b, ob_a,
                   ob_b, sem_ga, sem_gb, sem_wa, sem_wb):
    w = lax.axis_index("s") * NUM_CORES + lax.axis_index("c")
    iota = lax.iota(jnp.int32, 16)
    iota0 = iota * 0
    r_base = [iota + 16 * ilb for ilb in range(16)]
    u0 = w * D2PW
    pltpu.sync_copy(ids_hbm.at[pl.ds(u0 * 256, D2PW * 256)], idx_v)

    def gather_start(u, buf, sem):
        pltpu.async_copy(
            tbl_hbm.at[idx_v.at[pl.ds((u - u0) * 256, 256)]], buf, sem)

    def gather_wait(u, buf, sem):
        pltpu.make_async_copy(
            tbl_hbm.at[idx_v.at[pl.ds((u - u0) * 256, 256)]], buf, sem).wait()

    def write_start(u, buf, sem):
        j = u // IBP
        ibp = u % IBP
        pltpu.async_copy(
            buf.at[:, pl.ds(0, 2), :, pl.ds(0, 128)],
            out_hbm.at[j, :, pl.ds(ibp * 2, 2)], sem)

    def write_wait(u, buf, sem):
        j = u // IBP
        ibp = u % IBP
        pltpu.make_async_copy(
            buf.at[:, pl.ds(0, 2), :, pl.ds(0, 128)],
            out_hbm.at[j, :, pl.ds(ibp * 2, 2)], sem).wait()

    cb_lo = lax.shift_right_logical(iota, 3)
    cb_hi = cb_lo + 2
    cr_v = iota & 7

    def transpose_unit(src, dst):
        @plsc.parallel_loop(0, 256, unroll=8)
        def _(il):
            h_idx = iota0 + lax.shift_right_logical(il, 7)
            i_idx = iota0 + (il & 127)
            v_lo = src[il, pl.ds(0, 16)]
            v_hi = src[il, pl.ds(16, 16)]
            plsc.store_scatter(dst, [cb_lo, h_idx, cr_v, i_idx], v_lo)
            plsc.store_scatter(dst, [cb_hi, h_idx, cr_v, i_idx], v_hi)

    NS = D2PW // 2
    gather_start(u0, rows_a, sem_ga)

    def body(s, carry):
        ua = u0 + 2 * s
        ub = ua + 1
        gather_wait(ua, rows_a, sem_ga)
        gather_start(ub, rows_b, sem_gb)
        transpose_unit(rows_a, ob_a)

        @pl.when(s > 0)
        def _():
            write_wait(ua - 2, ob_a, sem_wa)

        write_start(ua, ob_a, sem_wa)
        gather_wait(ub, rows_b, sem_gb)

        @pl.when(s < NS - 1)
        def _():
            gather_start(ua + 2, rows_a, sem_ga)

        transpose_unit(rows_b, ob_b)

        @pl.when(s > 0)
        def _():
            write_wait(ub - 2, ob_b, sem_wb)

        write_start(ub, ob_b, sem_wb)
        return carry

    lax.fori_loop(0, NS, body, 0)
    write_wait(u0 + D2PW - 2, ob_a, sem_wa)
    write_wait(u0 + D2PW - 1, ob_b, sem_wb)


@jax.jit
def kernel(token_ids, embeddings):
    ids_t = token_ids.T.astype(jnp.int32)
    tbl_t = embeddings.T
    tail_flat = embeddings[FULL_TILES * 128:].reshape(-1)
    ids_lin, tbl_flat = _prep_kernel(ids_t, tbl_t, tail_flat)
    out5 = _gather_kernel(ids_lin, tbl_flat.reshape(VOCAB, EMBED_DIM))
    out = out5.transpose(2, 4, 0, 1, 3).reshape(NUM_BATCH, NUM_POS, EMBED_DIM)
    return out


# speedup vs baseline: 1.7920x; 1.2956x over previous
import functools

import jax
import jax.numpy as jnp
from jax import lax
from jax.experimental import pallas as pl
from jax.experimental.pallas import tpu as pltpu
from jax.experimental.pallas import tpu_sc as plsc

NUM_POS = 50
NUM_BATCH = 16384
NUM_TOKENS = NUM_BATCH * NUM_POS
VOCAB = 1000000
EMBED_DIM = 32
NUM_CORES = 2
NUM_SUBCORES = 16
NUM_WORKERS = NUM_CORES * NUM_SUBCORES

FULL_TILES = VOCAB // 128
TPW = FULL_TILES // NUM_WORKERS
DBL_TILES = FULL_TILES // 2
T2PW = DBL_TILES // NUM_WORKERS
EXTRA_T2 = NUM_WORKERS * T2PW
TAIL_COLS = VOCAB - FULL_TILES * 128

IBLOCKS = NUM_BATCH // 128
UNITS = NUM_POS * IBLOCKS
UPW = UNITS // NUM_WORKERS
DUNITS = UNITS // 2
D2PW = DUNITS // NUM_WORKERS
IBP = IBLOCKS // 2

_mesh = plsc.VectorSubcoreMesh(core_axis_name="c", subcore_axis_name="s")


@functools.partial(
    pl.kernel,
    mesh=_mesh,
    compiler_params=pltpu.CompilerParams(needs_layout_passes=False),
    out_type=(
        jax.ShapeDtypeStruct((NUM_TOKENS,), jnp.int32),
        jax.ShapeDtypeStruct((VOCAB * EMBED_DIM,), jnp.float32),
    ),
    scratch_types=[
        pltpu.VMEM((8, 2048), jnp.int32),
        pltpu.VMEM((32, 256), jnp.float32),
        pltpu.VMEM((32, 256), jnp.float32),
        pltpu.VMEM((8192,), jnp.float32),
        pltpu.VMEM((8192,), jnp.float32),
        pltpu.SemaphoreType.DMA,
        pltpu.SemaphoreType.DMA,
        pltpu.SemaphoreType.DMA,
        pltpu.SemaphoreType.DMA,
    ],
)
def _prep_kernel(ids_hbm, tbl_hbm, tail_hbm, ids_out, tbl_out, idsbuf,
                 cb_a, cb_b, tb_a, tb_b, sem_ra, sem_rb, sem_wa, sem_wb):
    w = lax.axis_index("s") * NUM_CORES + lax.axis_index("c")
    iota = lax.iota(jnp.int32, 16)
    iota0 = iota * 0
    c_lo = iota
    c_hi = iota + 16

    @pl.when(w < 28)
    def _():
        for k in range(2):
            su = w * 2 + k
            jb = su // 8
            cc = su % 8
            pltpu.sync_copy(
                ids_hbm.at[pl.ds(jb * 8, 8), pl.ds(cc * 2048, 2048)], idsbuf)
            for r in range(8):
                @pl.when(jb * 8 + r < NUM_POS)
                def _():
                    pltpu.sync_copy(
                        idsbuf.at[r],
                        ids_out.at[pl.ds((jb * 8 + r) * NUM_BATCH + cc * 2048,
                                         2048)])

    def read_start(t, buf, sem):
        for cb in range(4):
            pltpu.async_copy(
                tbl_hbm.at[pl.ds(cb * 8, 8), pl.ds(t * 256, 256)],
                buf.at[pl.ds(cb * 8, 8)], sem)

    def read_wait(t, buf, sem):
        for cb in range(4):
            pltpu.make_async_copy(
                tbl_hbm.at[pl.ds(cb * 8, 8), pl.ds(t * 256, 256)],
                buf.at[pl.ds(cb * 8, 8)], sem).wait()

    def write_start(t, buf, sem):
        pltpu.async_copy(buf, tbl_out.at[pl.ds(t * 8192, 8192)], sem)

    def write_wait(t, buf, sem):
        pltpu.make_async_copy(
            buf, tbl_out.at[pl.ds(t * 8192, 8192)], sem).wait()

    rot = [(iota + r) & 15 for r in range(16)]
    dvec = [((iota + r) & 15) * 32 + iota for r in range(16)]

    def transpose_tile(src, dst):
        @plsc.parallel_loop(0, 256, step=16, unroll=2)
        def _(y0):
            yb32 = y0 * 32
            for r in range(16):
                y_idx = rot[r] + y0
                d0 = dvec[r] + yb32
                v0 = plsc.load_gather(src, [c_lo, y_idx])
                plsc.store_scatter(dst, [d0], v0)
                v1 = plsc.load_gather(src, [c_hi, y_idx])
                plsc.store_scatter(dst, [d0 + 16], v1)

    t0 = w * T2PW
    NS = T2PW // 2
    read_start(t0, cb_a, sem_ra)

    def body(s, carry):
        ta = t0 + 2 * s
        tb = ta + 1
        read_wait(ta, cb_a, sem_ra)
        read_start(tb, cb_b, sem_rb)
        transpose_tile(cb_a, tb_a)

        @pl.when(s > 0)
        def _():
            write_wait(ta - 2, tb_a, sem_wa)

        write_start(ta, tb_a, sem_wa)
        read_wait(tb, cb_b, sem_rb)

        @pl.when(s < NS - 1)
        def _():
            read_start(ta + 2, cb_a, sem_ra)

        transpose_tile(cb_b, tb_b)

        @pl.when(s > 0)
        def _():
            write_wait(tb - 2, tb_b, sem_wb)

        write_start(tb, tb_b, sem_wb)
        return carry

    lax.fori_loop(0, NS, body, 0)
    write_wait(t0 + T2PW - 2, tb_a, sem_wa)
    write_wait(t0 + T2PW - 1, tb_b, sem_wb)

    @pl.when((w == 28) | (w == 29))
    def _():
        t = EXTRA_T2 + (w - 28)
        read_start(t, cb_a, sem_ra)
        read_wait(t, cb_a, sem_ra)
        transpose_tile(cb_a, tb_a)
        write_start(t, tb_a, sem_wa)
        write_wait(t, tb_a, sem_wa)

    @pl.when(w == 27)
    def _():
        pltpu.sync_copy(tail_hbm, tb_a.at[pl.ds(0, TAIL_COLS * EMBED_DIM)])
        pltpu.sync_copy(
            tb_a.at[pl.ds(0, TAIL_COLS * EMBED_DIM)],
            tbl_out.at[pl.ds(FULL_TILES * 128 * EMBED_DIM,
                             TAIL_COLS * EMBED_DIM)])


@functools.partial(
    pl.kernel,
    mesh=_mesh,
    compiler_params=pltpu.CompilerParams(
        use_tc_tiling_on_sc=False, needs_layout_passes=False),
    out_type=jax.ShapeDtypeStruct((NUM_POS, 4, IBLOCKS, 8, 128), jnp.float32),
    scratch_types=[
        pltpu.VMEM((UPW * 128,), jnp.int32),
        pltpu.VMEM((256, EMBED_DIM), jnp.float32),
        pltpu.VMEM((256, EMBED_DIM), jnp.float32),
        pltpu.VMEM((4, 3, 8, 129), jnp.float32),
        pltpu.VMEM((4, 3, 8, 129), jnp.float32),
        pltpu.SemaphoreType.DMA,
        pltpu.SemaphoreType.DMA,
        pltpu.SemaphoreType.DMA,
        pltpu.SemaphoreType.DMA,
    ],
)
def _gather_kernel(ids_hbm, tbl_hbm, out_hbm, idx_v, rows_a, rows_b, ob_a,
                   ob_b, sem_ga, sem_gb, sem_wa, sem_wb):
    w = lax.axis_index("s") * NUM_CORES + lax.axis_index("c")
    iota = lax.iota(jnp.int32, 16)
    iota0 = iota * 0
    r_base = [iota + 16 * ilb for ilb in range(16)]
    u0 = w * D2PW
    pltpu.sync_copy(ids_hbm.at[pl.ds(u0 * 256, D2PW * 256)], idx_v)

    def gather_start(u, buf, sem):
        pltpu.async_copy(
            tbl_hbm.at[idx_v.at[pl.ds((u - u0) * 256, 256)]], buf, sem)

    def gather_wait(u, buf, sem):
        pltpu.make_async_copy(
            tbl_hbm.at[idx_v.at[pl.ds((u - u0) * 256, 256)]], buf, sem).wait()

    def write_start(u, buf, sem):
        j = u // IBP
        ibp = u % IBP
        pltpu.async_copy(
            buf.at[:, pl.ds(0, 2), :, pl.ds(0, 128)],
            out_hbm.at[j, :, pl.ds(ibp * 2, 2)], sem)

    def write_wait(u, buf, sem):
        j = u // IBP
        ibp = u % IBP
        pltpu.make_async_copy(
            buf.at[:, pl.ds(0, 2), :, pl.ds(0, 128)],
            out_hbm.at[j, :, pl.ds(ibp * 2, 2)], sem).wait()

    cb_lo = lax.shift_right_logical(iota, 3)
    cb_hi = cb_lo + 2
    cr_v = iota & 7

    def transpose_unit(src, dst):
        @plsc.parallel_loop(0, 256, unroll=8)
        def _(il):
            h_idx = iota0 + lax.shift_right_logical(il, 7)
            i_idx = iota0 + (il & 127)
            v_lo = src[il, pl.ds(0, 16)]
            v_hi = src[il, pl.ds(16, 16)]
            plsc.store_scatter(dst, [cb_lo, h_idx, cr_v, i_idx], v_lo)
            plsc.store_scatter(dst, [cb_hi, h_idx, cr_v, i_idx], v_hi)

    NS = D2PW // 2
    gather_start(u0, rows_a, sem_ga)

    def body(s, carry):
        ua = u0 + 2 * s
        ub = ua + 1
        gather_wait(ua, rows_a, sem_ga)
        gather_start(ub, rows_b, sem_gb)
        transpose_unit(rows_a, ob_a)

        @pl.when(s > 0)
        def _():
            write_wait(ua - 2, ob_a, sem_wa)

        write_start(ua, ob_a, sem_wa)
        gather_wait(ub, rows_b, sem_gb)

        @pl.when(s < NS - 1)
        def _():
            gather_start(ua + 2, rows_a, sem_ga)

        transpose_unit(rows_b, ob_b)

        @pl.when(s > 0)
        def _():
            write_wait(ub - 2, ob_b, sem_wb)

        write_start(ub, ob_b, sem_wb)
        return carry

    lax.fori_loop(0, NS, body, 0)
    write_wait(u0 + D2PW - 2, ob_a, sem_wa)
    write_wait(u0 + D2PW - 1, ob_b, sem_wb)


@jax.jit
def kernel(token_ids, embeddings):
    ids_t = token_ids.T.astype(jnp.int32)
    tbl_t = embeddings.T
    tail_flat = embeddings[FULL_TILES * 128:].reshape(-1)
    ids_lin, tbl_flat = _prep_kernel(ids_t, tbl_t, tail_flat)
    out5 = _gather_kernel(ids_lin, tbl_flat.reshape(VOCAB, EMBED_DIM))
    out = out5.transpose(2, 4, 0, 1, 3).reshape(NUM_BATCH, NUM_POS, EMBED_DIM)
    return out


# speedup vs baseline: 1.8837x; 1.0512x over previous
import functools

import jax
import jax.numpy as jnp
from jax import lax
from jax.experimental import pallas as pl
from jax.experimental.pallas import tpu as pltpu
from jax.experimental.pallas import tpu_sc as plsc

NUM_POS = 50
NUM_BATCH = 16384
NUM_TOKENS = NUM_BATCH * NUM_POS
VOCAB = 1000000
EMBED_DIM = 32
NUM_CORES = 2
NUM_SUBCORES = 16
NUM_WORKERS = NUM_CORES * NUM_SUBCORES

FULL_TILES = VOCAB // 128
TPW = FULL_TILES // NUM_WORKERS
DBL_TILES = FULL_TILES // 2
T2PW = DBL_TILES // NUM_WORKERS
EXTRA_T2 = NUM_WORKERS * T2PW
TAIL_COLS = VOCAB - FULL_TILES * 128

IBLOCKS = NUM_BATCH // 128
UNITS = NUM_POS * IBLOCKS
UPW = UNITS // NUM_WORKERS
DUNITS = UNITS // 2
D2PW = DUNITS // NUM_WORKERS
IBP = IBLOCKS // 2

_mesh = plsc.VectorSubcoreMesh(core_axis_name="c", subcore_axis_name="s")


@functools.partial(
    pl.kernel,
    mesh=_mesh,
    compiler_params=pltpu.CompilerParams(needs_layout_passes=False),
    out_type=(
        jax.ShapeDtypeStruct((NUM_TOKENS,), jnp.int32),
        jax.ShapeDtypeStruct((VOCAB * EMBED_DIM,), jnp.float32),
    ),
    scratch_types=[
        pltpu.VMEM((8, 2048), jnp.int32),
        pltpu.VMEM((32, 256), jnp.float32),
        pltpu.VMEM((32, 256), jnp.float32),
        pltpu.VMEM((8192,), jnp.float32),
        pltpu.VMEM((8192,), jnp.float32),
        pltpu.SemaphoreType.DMA,
        pltpu.SemaphoreType.DMA,
        pltpu.SemaphoreType.DMA,
        pltpu.SemaphoreType.DMA,
    ],
)
def _prep_kernel(ids_hbm, tbl_hbm, tail_hbm, ids_out, tbl_out, idsbuf,
                 cb_a, cb_b, tb_a, tb_b, sem_ra, sem_rb, sem_wa, sem_wb):
    w = lax.axis_index("s") * NUM_CORES + lax.axis_index("c")
    iota = lax.iota(jnp.int32, 16)
    iota0 = iota * 0
    c_lo = iota
    c_hi = iota + 16

    @pl.when(w < 28)
    def _():
        for k in range(2):
            su = w * 2 + k
            jb = su // 8
            cc = su % 8
            pltpu.sync_copy(
                ids_hbm.at[pl.ds(jb * 8, 8), pl.ds(cc * 2048, 2048)], idsbuf)
            for r in range(8):
                @pl.when(jb * 8 + r < NUM_POS)
                def _():
                    pltpu.sync_copy(
                        idsbuf.at[r],
                        ids_out.at[pl.ds((jb * 8 + r) * NUM_BATCH + cc * 2048,
                                         2048)])

    def read_start(t, buf, sem):
        for cb in range(4):
            pltpu.async_copy(
                tbl_hbm.at[pl.ds(cb * 8, 8), pl.ds(t * 256, 256)],
                buf.at[pl.ds(cb * 8, 8)], sem)

    def read_wait(t, buf, sem):
        for cb in range(4):
            pltpu.make_async_copy(
                tbl_hbm.at[pl.ds(cb * 8, 8), pl.ds(t * 256, 256)],
                buf.at[pl.ds(cb * 8, 8)], sem).wait()

    def write_start(t, buf, sem):
        pltpu.async_copy(buf, tbl_out.at[pl.ds(t * 8192, 8192)], sem)

    def write_wait(t, buf, sem):
        pltpu.make_async_copy(
            buf, tbl_out.at[pl.ds(t * 8192, 8192)], sem).wait()

    rot = [(iota + r) & 15 for r in range(16)]
    dvec = [((iota + r) & 15) * 32 + iota for r in range(16)]

    def transpose_tile(src, dst):
        @plsc.parallel_loop(0, 256, step=16, unroll=2)
        def _(y0):
            yb32 = y0 * 32
            for r in range(16):
                y_idx = rot[r] + y0
                d0 = dvec[r] + yb32
                v0 = plsc.load_gather(src, [c_lo, y_idx])
                plsc.store_scatter(dst, [d0], v0)
                v1 = plsc.load_gather(src, [c_hi, y_idx])
                plsc.store_scatter(dst, [d0 + 16], v1)

    t0 = w * T2PW
    NS = T2PW // 2
    read_start(t0, cb_a, sem_ra)

    def body(s, carry):
        ta = t0 + 2 * s
        tb = ta + 1
        read_start(tb, cb_b, sem_rb)

        @pl.when(s > 0)
        def _():
            write_wait(ta - 2, tb_a, sem_wa)

        read_wait(ta, cb_a, sem_ra)
        transpose_tile(cb_a, tb_a)

        @pl.when(s < NS - 1)
        def _():
            read_start(ta + 2, cb_a, sem_ra)

        write_start(ta, tb_a, sem_wa)

        @pl.when(s > 0)
        def _():
            write_wait(tb - 2, tb_b, sem_wb)

        read_wait(tb, cb_b, sem_rb)
        transpose_tile(cb_b, tb_b)
        write_start(tb, tb_b, sem_wb)
        return carry

    lax.fori_loop(0, NS, body, 0)
    write_wait(t0 + T2PW - 2, tb_a, sem_wa)
    write_wait(t0 + T2PW - 1, tb_b, sem_wb)

    @pl.when((w == 28) | (w == 29))
    def _():
        t = EXTRA_T2 + (w - 28)
        read_start(t, cb_a, sem_ra)
        read_wait(t, cb_a, sem_ra)
        transpose_tile(cb_a, tb_a)
        write_start(t, tb_a, sem_wa)
        write_wait(t, tb_a, sem_wa)

    @pl.when(w == 27)
    def _():
        pltpu.sync_copy(tail_hbm, tb_a.at[pl.ds(0, TAIL_COLS * EMBED_DIM)])
        pltpu.sync_copy(
            tb_a.at[pl.ds(0, TAIL_COLS * EMBED_DIM)],
            tbl_out.at[pl.ds(FULL_TILES * 128 * EMBED_DIM,
                             TAIL_COLS * EMBED_DIM)])


@functools.partial(
    pl.kernel,
    mesh=_mesh,
    compiler_params=pltpu.CompilerParams(
        use_tc_tiling_on_sc=False, needs_layout_passes=False),
    out_type=jax.ShapeDtypeStruct((NUM_POS, 4, IBLOCKS, 8, 128), jnp.float32),
    scratch_types=[
        pltpu.VMEM((UPW * 128,), jnp.int32),
        pltpu.VMEM((256, EMBED_DIM), jnp.float32),
        pltpu.VMEM((256, EMBED_DIM), jnp.float32),
        pltpu.VMEM((4, 3, 8, 129), jnp.float32),
        pltpu.VMEM((4, 3, 8, 129), jnp.float32),
        pltpu.SemaphoreType.DMA,
        pltpu.SemaphoreType.DMA,
        pltpu.SemaphoreType.DMA,
        pltpu.SemaphoreType.DMA,
    ],
)
def _gather_kernel(ids_hbm, tbl_hbm, out_hbm, idx_v, rows_a, rows_b, ob_a,
                   ob_b, sem_ga, sem_gb, sem_wa, sem_wb):
    w = lax.axis_index("s") * NUM_CORES + lax.axis_index("c")
    iota = lax.iota(jnp.int32, 16)
    iota0 = iota * 0
    r_base = [iota + 16 * ilb for ilb in range(16)]
    u0 = w * D2PW
    pltpu.sync_copy(ids_hbm.at[pl.ds(u0 * 256, D2PW * 256)], idx_v)

    def gather_start(u, buf, sem):
        pltpu.async_copy(
            tbl_hbm.at[idx_v.at[pl.ds((u - u0) * 256, 256)]], buf, sem)

    def gather_wait(u, buf, sem):
        pltpu.make_async_copy(
            tbl_hbm.at[idx_v.at[pl.ds((u - u0) * 256, 256)]], buf, sem).wait()

    def write_start(u, buf, sem):
        j = u // IBP
        ibp = u % IBP
        pltpu.async_copy(
            buf.at[:, pl.ds(0, 2), :, pl.ds(0, 128)],
            out_hbm.at[j, :, pl.ds(ibp * 2, 2)], sem)

    def write_wait(u, buf, sem):
        j = u // IBP
        ibp = u % IBP
        pltpu.make_async_copy(
            buf.at[:, pl.ds(0, 2), :, pl.ds(0, 128)],
            out_hbm.at[j, :, pl.ds(ibp * 2, 2)], sem).wait()

    cb_lo = lax.shift_right_logical(iota, 3)
    cb_hi = cb_lo + 2
    cr_v = iota & 7

    def transpose_unit(src, dst):
        @plsc.parallel_loop(0, 256, unroll=8)
        def _(il):
            h_idx = iota0 + lax.shift_right_logical(il, 7)
            i_idx = iota0 + (il & 127)
            v_lo = src[il, pl.ds(0, 16)]
            v_hi = src[il, pl.ds(16, 16)]
            plsc.store_scatter(dst, [cb_lo, h_idx, cr_v, i_idx], v_lo)
            plsc.store_scatter(dst, [cb_hi, h_idx, cr_v, i_idx], v_hi)

    NS = D2PW // 2
    gather_start(u0, rows_a, sem_ga)

    def body(s, carry):
        ua = u0 + 2 * s
        ub = ua + 1
        gather_start(ub, rows_b, sem_gb)

        @pl.when(s > 0)
        def _():
            write_wait(ua - 2, ob_a, sem_wa)

        gather_wait(ua, rows_a, sem_ga)
        transpose_unit(rows_a, ob_a)

        @pl.when(s < NS - 1)
        def _():
            gather_start(ua + 2, rows_a, sem_ga)

        write_start(ua, ob_a, sem_wa)

        @pl.when(s > 0)
        def _():
            write_wait(ub - 2, ob_b, sem_wb)

        gather_wait(ub, rows_b, sem_gb)
        transpose_unit(rows_b, ob_b)
        write_start(ub, ob_b, sem_wb)
        return carry

    lax.fori_loop(0, NS, body, 0)
    write_wait(u0 + D2PW - 2, ob_a, sem_wa)
    write_wait(u0 + D2PW - 1, ob_b, sem_wb)


@jax.jit
def kernel(token_ids, embeddings):
    ids_t = token_ids.T.astype(jnp.int32)
    tbl_t = embeddings.T
    tail_flat = embeddings[FULL_TILES * 128:].reshape(-1)
    ids_lin, tbl_flat = _prep_kernel(ids_t, tbl_t, tail_flat)
    out5 = _gather_kernel(ids_lin, tbl_flat.reshape(VOCAB, EMBED_DIM))
    out = out5.transpose(2, 4, 0, 1, 3).reshape(NUM_BATCH, NUM_POS, EMBED_DIM)
    return out
